# Initial kernel scaffold; baseline (speedup 1.0000x reference)
import jax
import jax.numpy as jnp
from jax.experimental import pallas as pl


def kernel(emb, W1, b1, W2, W1n, b1n, W2n, h_bias, ln1_g, ln1_b, ln2_g, ln2_b, edge_index, edge_type):
    raise NotImplementedError("write your pallas kernel here")


# speedup vs baseline: 5.3383x; 5.3383x over previous
import functools

import jax
import jax.numpy as jnp
from jax import lax
from jax.experimental import pallas as pl
from jax.experimental.pallas import tpu as pltpu
from jax.experimental.pallas import tpu_sc as plsc

_N_USER = 6000
_N_NODES = 10000
_HID = 64
_MEM = 4
_RELS = 5
_E = 160000
_EPS = 1e-5

_NC = 2
_NS = 16
_NP = 10112
_QROWS = _RELS * _NP
_UP = 6016
_CH = 128


def _leaky(x):
    return jnp.where(x >= 0, x, 0.2 * x)


_EPT_A = _E // _NS
_NF_A = _EPT_A // _CH
_TL_A = _EPT_A - _NF_A * _CH
_RPT_A = _QROWS // _NS
_DPT_A = _NP // _NS


def _make_phase_a():
    mesh = plsc.VectorSubcoreMesh(
        core_axis_name="c", subcore_axis_name="s",
        num_cores=_NC, num_subcores=_NS)

    @functools.partial(
        pl.kernel,
        out_type=(jax.ShapeDtypeStruct((_NC, _QROWS, 32), jnp.float32),
                  jax.ShapeDtypeStruct((_NP, 8), jnp.float32)),
        mesh=mesh,
        compiler_params=pltpu.CompilerParams(use_tc_tiling_on_sc=False),
        scratch_types=[
            pltpu.VMEM_SHARED((_QROWS, 32), jnp.float32),
            pltpu.VMEM_SHARED((_NP, 8), jnp.float32),
            pltpu.VMEM((_CH,), jnp.int32),
            pltpu.VMEM((_CH,), jnp.int32),
            pltpu.VMEM((_CH,), jnp.int32),
            pltpu.VMEM((_CH, 32), jnp.float32),
            pltpu.VMEM((_TL_A,), jnp.int32),
            pltpu.VMEM((_TL_A,), jnp.int32),
            pltpu.VMEM((_TL_A,), jnp.int32),
            pltpu.VMEM((_TL_A, 32), jnp.float32),
            pltpu.VMEM((_CH, 32), jnp.float32),
            pltpu.VMEM((_CH, 8), jnp.float32),
            pltpu.VMEM((_CH, 8), jnp.float32),
            pltpu.SemaphoreType.DMA,
        ],
    )
    def phase_a(embL, embR, srcv, qv, dstv, zeros32, zeros8, ones8,
                xagg_out, deg_out,
                acc, dacc, src_v, q_v, dst_v, rows_v,
                src_t, q_t, dst_t, rows_t, zb32, zb8, ones_v, sem):
        c = lax.axis_index("c")
        s = lax.axis_index("s")

        pltpu.sync_copy(zeros32, zb32)
        pltpu.sync_copy(zeros8, zb8)
        pltpu.sync_copy(ones8, ones_v)

        r0 = s * _RPT_A
        nz = _RPT_A // _CH
        def _za(i, carry):
            pltpu.sync_copy(zb32, acc.at[pl.ds(r0 + i * _CH, _CH)])
            return carry
        lax.fori_loop(0, nz, _za, 0)
        rem = _RPT_A - nz * _CH
        if rem:
            pltpu.sync_copy(zb32.at[pl.ds(0, rem)],
                            acc.at[pl.ds(r0 + nz * _CH, rem)])

        @pl.when(c == 0)
        def _zero_deg():
            d0 = s * _DPT_A
            ndz = _DPT_A // _CH
            def _zd(i, carry):
                pltpu.sync_copy(zb8, dacc.at[pl.ds(d0 + i * _CH, _CH)])
                return carry
            lax.fori_loop(0, ndz, _zd, 0)
            drem = _DPT_A - ndz * _CH
            if drem:
                pltpu.sync_copy(zb8.at[pl.ds(0, drem)],
                                dacc.at[pl.ds(d0 + ndz * _CH, drem)])

        plsc.subcore_barrier()

        eb = s * _EPT_A

        def _chunk(j, carry):
            off = eb + j * _CH
            pltpu.sync_copy(srcv.at[pl.ds(off, _CH)], src_v)
            pltpu.sync_copy(qv.at[pl.ds(off, _CH)], q_v)

            @pl.when(c == 0)
            def _g0():
                pltpu.async_copy(embL.at[src_v], rows_v, sem).wait()

            @pl.when(c == 1)
            def _g1():
                pltpu.async_copy(embR.at[src_v], rows_v, sem).wait()

            pltpu.sync_copy(rows_v, acc.at[q_v], add=True)

            @pl.when(c == 0)
            def _deg():
                pltpu.sync_copy(dstv.at[pl.ds(off, _CH)], dst_v)
                pltpu.sync_copy(ones_v, dacc.at[dst_v], add=True)
            return carry

        lax.fori_loop(0, _NF_A, _chunk, 0)

        offt = eb + _NF_A * _CH
        pltpu.sync_copy(srcv.at[pl.ds(offt, _TL_A)], src_t)
        pltpu.sync_copy(qv.at[pl.ds(offt, _TL_A)], q_t)

        @pl.when(c == 0)
        def _gt0():
            pltpu.async_copy(embL.at[src_t], rows_t, sem).wait()

        @pl.when(c == 1)
        def _gt1():
            pltpu.async_copy(embR.at[src_t], rows_t, sem).wait()

        pltpu.sync_copy(rows_t, acc.at[q_t], add=True)

        @pl.when(c == 0)
        def _degt():
            pltpu.sync_copy(dstv.at[pl.ds(offt, _TL_A)], dst_t)
            pltpu.sync_copy(ones_v.at[pl.ds(0, _TL_A)], dacc.at[dst_t],
                            add=True)

        plsc.subcore_barrier()

        pltpu.sync_copy(acc.at[pl.ds(s * _RPT_A, _RPT_A)],
                        xagg_out.at[c, pl.ds(s * _RPT_A, _RPT_A)])

        @pl.when(c == 0)
        def _wdeg():
            pltpu.sync_copy(dacc.at[pl.ds(s * _DPT_A, _DPT_A)],
                            deg_out.at[pl.ds(s * _DPT_A, _DPT_A)])

    return phase_a


_EPT_C = _E // (_NC * _NS)
_NF_C = _EPT_C // _CH
_TL_C = _EPT_C - _NF_C * _CH
_RPT_C = _UP // _NS


def _make_phase_c():
    mesh = plsc.VectorSubcoreMesh(
        core_axis_name="c", subcore_axis_name="s",
        num_cores=_NC, num_subcores=_NS)

    @functools.partial(
        pl.kernel,
        out_type=(jax.ShapeDtypeStruct((_NC, _UP, 128), jnp.float32),
                  jax.ShapeDtypeStruct((_NC, _UP, 8), jnp.float32)),
        mesh=mesh,
        compiler_params=pltpu.CompilerParams(use_tc_tiling_on_sc=False),
        scratch_types=[
            pltpu.VMEM_SHARED((_UP, 128), jnp.float32),
            pltpu.VMEM_SHARED((_UP, 8), jnp.float32),
            pltpu.VMEM((_CH,), jnp.int32),
            pltpu.VMEM((_CH,), jnp.int32),
            pltpu.VMEM((_CH, 128), jnp.float32),
            pltpu.VMEM((_TL_C,), jnp.int32),
            pltpu.VMEM((_TL_C,), jnp.int32),
            pltpu.VMEM((_TL_C, 128), jnp.float32),
            pltpu.VMEM((_CH, 128), jnp.float32),
            pltpu.VMEM((_CH, 8), jnp.float32),
            pltpu.VMEM((_CH, 8), jnp.float32),
            pltpu.SemaphoreType.DMA,
        ],
    )
    def phase_c(xc, srcv, cuv, zeros128, zeros8, ones8,
                sum_out, cnt_out,
                acc, cacc, src_v, cu_v, rows_v, src_t, cu_t, rows_t,
                zb128, zb8, ones_v, sem):
        c = lax.axis_index("c")
        s = lax.axis_index("s")

        pltpu.sync_copy(zeros128, zb128)
        pltpu.sync_copy(zeros8, zb8)
        pltpu.sync_copy(ones8, ones_v)

        r0 = s * _RPT_C
        nz = _RPT_C // _CH
        def _za(i, carry):
            pltpu.sync_copy(zb128, acc.at[pl.ds(r0 + i * _CH, _CH)])
            pltpu.sync_copy(zb8, cacc.at[pl.ds(r0 + i * _CH, _CH)])
            return carry
        lax.fori_loop(0, nz, _za, 0)
        rem = _RPT_C - nz * _CH
        if rem:
            pltpu.sync_copy(zb128.at[pl.ds(0, rem)],
                            acc.at[pl.ds(r0 + nz * _CH, rem)])
            pltpu.sync_copy(zb8.at[pl.ds(0, rem)],
                            cacc.at[pl.ds(r0 + nz * _CH, rem)])

        plsc.subcore_barrier()

        eb = c * (_E // _NC) + s * _EPT_C

        def _chunk(j, carry):
            off = eb + j * _CH
            pltpu.sync_copy(srcv.at[pl.ds(off, _CH)], src_v)
            pltpu.sync_copy(cuv.at[pl.ds(off, _CH)], cu_v)
            pltpu.async_copy(xc.at[src_v], rows_v, sem).wait()
            pltpu.sync_copy(rows_v, acc.at[cu_v], add=True)
            pltpu.sync_copy(ones_v, cacc.at[cu_v], add=True)
            return carry

        lax.fori_loop(0, _NF_C, _chunk, 0)

        offt = eb + _NF_C * _CH
        pltpu.sync_copy(srcv.at[pl.ds(offt, _TL_C)], src_t)
        pltpu.sync_copy(cuv.at[pl.ds(offt, _TL_C)], cu_t)
        pltpu.async_copy(xc.at[src_t], rows_t, sem).wait()
        pltpu.sync_copy(rows_t, acc.at[cu_t], add=True)
        pltpu.sync_copy(ones_v.at[pl.ds(0, _TL_C)], cacc.at[cu_t], add=True)

        plsc.subcore_barrier()

        pltpu.sync_copy(acc.at[pl.ds(s * _RPT_C, _RPT_C)],
                        sum_out.at[c, pl.ds(s * _RPT_C, _RPT_C)])
        pltpu.sync_copy(cacc.at[pl.ds(s * _RPT_C, _RPT_C)],
                        cnt_out.at[c, pl.ds(s * _RPT_C, _RPT_C)])

    return phase_c


_BB = 1000


def _pb_body(emb_ref, xaL_ref, xaR_ref, deg_ref, w1f_ref, b1f_ref, w2f_ref,
             w1n_ref, b1n_ref, w2nf_ref, hb_ref, g1_ref, bb1_ref,
             g2_ref, bb2_ref, out_ref):
    x = emb_ref[...]
    cf = _leaky(jax.lax.dot(x, w1f_ref[...],
                            preferred_element_type=jnp.float32)
                + b1f_ref[...])
    deg = jnp.maximum(deg_ref[:, 0:1], 1.0)

    agg = jnp.zeros((_BB, _HID), jnp.float32)
    for r in range(_RELS):
        xa = jnp.concatenate([xaL_ref[r], xaR_ref[r]], axis=-1)
        s2 = jax.lax.dot(xa, w2f_ref[r],
                         preferred_element_type=jnp.float32)
        for m in range(_MEM):
            agg += cf[:, r * _MEM + m:r * _MEM + m + 1] * \
                s2[:, m * _HID:(m + 1) * _HID]
    agg = agg / deg

    mu = jnp.mean(agg, axis=-1, keepdims=True)
    var = jnp.mean((agg - mu) ** 2, axis=-1, keepdims=True)
    nr = (agg - mu) * jax.lax.rsqrt(var + _EPS) * g1_ref[...] \
        + bb1_ref[...] + hb_ref[...]

    cn = _leaky(jax.lax.dot(x, w1n_ref[...],
                            preferred_element_type=jnp.float32)
                + b1n_ref[...])
    t = jax.lax.dot(x, w2nf_ref[...],
                    preferred_element_type=jnp.float32)
    for m in range(_MEM):
        nr += cn[:, m:m + 1] * t[:, m * _HID:(m + 1) * _HID]
    x1 = _leaky(nr)

    xcat = jnp.concatenate([x, x1], axis=-1)
    mu2 = jnp.mean(xcat, axis=-1, keepdims=True)
    var2 = jnp.mean((xcat - mu2) ** 2, axis=-1, keepdims=True)
    out_ref[...] = (xcat - mu2) * jax.lax.rsqrt(var2 + _EPS) * g2_ref[...] \
        + bb2_ref[...]


def _phase_b(emb, xaL, xaR, deg, w1f, b1f, w2f, w1n, b1n, w2nf,
             hb, g1, bb1, g2, bb2):
    grid = (_N_NODES // _BB,)
    return pl.pallas_call(
        _pb_body,
        grid=grid,
        in_specs=[
            pl.BlockSpec((_BB, _HID), lambda i: (i, 0)),
            pl.BlockSpec((_RELS, _BB, 32), lambda i: (0, i, 0)),
            pl.BlockSpec((_RELS, _BB, 32), lambda i: (0, i, 0)),
            pl.BlockSpec((_BB, 8), lambda i: (i, 0)),
            pl.BlockSpec((_HID, _RELS * _MEM), lambda i: (0, 0)),
            pl.BlockSpec((1, _RELS * _MEM), lambda i: (0, 0)),
            pl.BlockSpec((_RELS, _HID, _MEM * _HID), lambda i: (0, 0, 0)),
            pl.BlockSpec((_HID, _MEM), lambda i: (0, 0)),
            pl.BlockSpec((1, _MEM), lambda i: (0, 0)),
            pl.BlockSpec((_HID, _MEM * _HID), lambda i: (0, 0)),
            pl.BlockSpec((1, _HID), lambda i: (0, 0)),
            pl.BlockSpec((1, _HID), lambda i: (0, 0)),
            pl.BlockSpec((1, _HID), lambda i: (0, 0)),
            pl.BlockSpec((1, 2 * _HID), lambda i: (0, 0)),
            pl.BlockSpec((1, 2 * _HID), lambda i: (0, 0)),
        ],
        out_specs=pl.BlockSpec((_BB, 2 * _HID), lambda i: (i, 0)),
        out_shape=jax.ShapeDtypeStruct((_N_NODES, 2 * _HID), jnp.float32),
    )(emb, xaL, xaR, deg, w1f, b1f, w2f, w1n, b1n, w2nf, hb, g1, bb1, g2, bb2)


_BD = 1000


def _pd_body(xc_ref, s_ref, c_ref, out_ref):
    cnt = c_ref[0, :, 0:1] + c_ref[1, :, 0:1] + 1.0
    out_ref[...] = (s_ref[0] + s_ref[1] + xc_ref[...]) / cnt


def _phase_d(xc, sums, cnts):
    grid = (_N_USER // _BD,)
    return pl.pallas_call(
        _pd_body,
        grid=grid,
        in_specs=[
            pl.BlockSpec((_BD, 2 * _HID), lambda i: (i, 0)),
            pl.BlockSpec((_NC, _BD, 2 * _HID), lambda i: (0, i, 0)),
            pl.BlockSpec((_NC, _BD, 8), lambda i: (0, i, 0)),
        ],
        out_specs=pl.BlockSpec((_BD, 2 * _HID), lambda i: (i, 0)),
        out_shape=jax.ShapeDtypeStruct((_N_USER, 2 * _HID), jnp.float32),
    )(xc, sums, cnts)


_make_phase_a = functools.cache(_make_phase_a)
_make_phase_c = functools.cache(_make_phase_c)


def kernel(emb, W1, b1, W2, W1n, b1n, W2n, h_bias, ln1_g, ln1_b,
           ln2_g, ln2_b, edge_index, edge_type):
    src = edge_index[0].astype(jnp.int32)
    dst = edge_index[1].astype(jnp.int32)
    et = edge_type.astype(jnp.int32)

    q = et * _NP + dst
    mask = (et == 0) & (src < _N_USER) & (dst < _N_USER)
    cu = jnp.where(mask, dst, _N_USER)

    embL = emb[:, :32]
    embR = emb[:, 32:]
    zeros32 = jnp.zeros((_CH, 32), jnp.float32)
    zeros8 = jnp.zeros((_CH, 8), jnp.float32)
    zeros128 = jnp.zeros((_CH, 128), jnp.float32)
    ones8 = jnp.ones((_CH, 8), jnp.float32)

    xagg, deg = _make_phase_a()(embL, embR, src, q, dst, zeros32, zeros8,
                                ones8)
    xaL = xagg[0].reshape(_RELS, _NP, 32)
    xaR = xagg[1].reshape(_RELS, _NP, 32)

    w1f = W1.transpose(1, 0, 2).reshape(_HID, _RELS * _MEM)
    b1f = b1.reshape(1, _RELS * _MEM)
    w2f = W2.transpose(0, 3, 1, 2).reshape(_RELS, _HID, _MEM * _HID)
    w2nf = W2n.transpose(2, 0, 1).reshape(_HID, _MEM * _HID)
    b1nf = b1n.reshape(1, _MEM)
    hb = h_bias.reshape(1, _HID)
    g1 = ln1_g.reshape(1, _HID)
    bb1 = ln1_b.reshape(1, _HID)
    g2 = ln2_g.reshape(1, 2 * _HID)
    bb2 = ln2_b.reshape(1, 2 * _HID)

    xc = _phase_b(emb, xaL, xaR, deg, w1f, b1f, w2f, W1n, b1nf, w2nf,
                  hb, g1, bb1, g2, bb2)

    sums, cnts = _make_phase_c()(xc, src, cu, zeros128, zeros8, ones8)
    user_pool = _phase_d(xc, sums, cnts)

    return (xc, user_pool)


# speedup vs baseline: 6.8859x; 1.2899x over previous
import functools

import jax
import jax.numpy as jnp
from jax import lax
from jax.experimental import pallas as pl
from jax.experimental.pallas import tpu as pltpu
from jax.experimental.pallas import tpu_sc as plsc

_N_USER = 6000
_N_NODES = 10000
_HID = 64
_MEM = 4
_RELS = 5
_E = 160000
_EPS = 1e-5

_NC = 2
_NS = 16
_NP = 10112
_QROWS = _RELS * _NP
_UP = 6016
_CH = 128


def _leaky(x):
    return jnp.where(x >= 0, x, 0.2 * x)


_NROW = _E // _CH
_RPT_MAIN_A = 78
_NB_A = 3
_NSUP_A = _RPT_MAIN_A // _NB_A
_LEFT_A = _NROW - _RPT_MAIN_A * _NS
_RPT_A = _QROWS // _NS
_DPT_A = _NP // _NS


def _make_phase_a():
    mesh = plsc.VectorSubcoreMesh(
        core_axis_name="c", subcore_axis_name="s",
        num_cores=_NC, num_subcores=_NS)

    @functools.partial(
        pl.kernel,
        out_type=(jax.ShapeDtypeStruct((_NC, _QROWS, 32), jnp.float32),
                  jax.ShapeDtypeStruct((_NC, _NP, 8), jnp.float32)),
        mesh=mesh,
        compiler_params=pltpu.CompilerParams(use_tc_tiling_on_sc=False),
        scratch_types=[
            pltpu.VMEM_SHARED((_QROWS, 32), jnp.float32),
            pltpu.VMEM_SHARED((_NP, 8), jnp.float32),
            pltpu.VMEM((_NB_A, 3, _CH), jnp.int32),
            pltpu.VMEM((_NB_A, _CH, 32), jnp.float32),
            pltpu.VMEM((_CH, 8), jnp.float32),
            pltpu.VMEM((_CH, 8), jnp.float32),
            pltpu.SemaphoreType.DMA,
            pltpu.SemaphoreType.DMA,
        ],
    )
    def phase_a(embL, embR, sqd, zeros32, zeros8, ones8,
                xagg_out, deg_out,
                acc, dacc, idxb, rows, zb8, ones_v, sem_g, sem_s):
        c = lax.axis_index("c")
        s = lax.axis_index("s")
        do_deg = jnp.where(c == 0, s < 8, s >= 8)

        zb32 = rows.at[0]
        pltpu.sync_copy(zeros32, zb32)
        pltpu.sync_copy(zeros8, zb8)
        pltpu.sync_copy(ones8, ones_v)

        r0 = s * _RPT_A
        nz = _RPT_A // _CH
        def _za(i, carry):
            pltpu.sync_copy(zb32, acc.at[pl.ds(r0 + i * _CH, _CH)])
            return carry
        lax.fori_loop(0, nz, _za, 0)
        rem = _RPT_A - nz * _CH
        if rem:
            pltpu.sync_copy(zb32.at[pl.ds(0, rem)],
                            acc.at[pl.ds(r0 + nz * _CH, rem)])

        d0 = s * _DPT_A
        ndz = _DPT_A // _CH
        def _zd(i, carry):
            pltpu.sync_copy(zb8, dacc.at[pl.ds(d0 + i * _CH, _CH)])
            return carry
        lax.fori_loop(0, ndz, _zd, 0)
        drem = _DPT_A - ndz * _CH
        if drem:
            pltpu.sync_copy(zb8.at[pl.ds(0, drem)],
                            dacc.at[pl.ds(d0 + ndz * _CH, drem)])

        plsc.subcore_barrier()

        row_base = s * _RPT_MAIN_A
        pltpu.sync_copy(sqd.at[pl.ds(row_base, _NB_A)], idxb)

        def _emb_gather(b, dst_rows):
            @pl.when(c == 0)
            def _g0():
                pltpu.async_copy(embL.at[idxb.at[b, 0]], dst_rows, sem_g)

            @pl.when(c == 1)
            def _g1():
                pltpu.async_copy(embR.at[idxb.at[b, 0]], dst_rows, sem_g)

        def _super(k, carry):
            for b in range(_NB_A):
                _emb_gather(b, rows.at[b])
            for b in range(_NB_A):
                pltpu.make_async_copy(embL.at[idxb.at[b, 0]], rows.at[b],
                                      sem_g).wait()
            for b in range(_NB_A):
                pltpu.async_copy(rows.at[b], acc.at[idxb.at[b, 1]], sem_s,
                                 add=True)

            @pl.when(do_deg)
            def _deg():
                for b in range(_NB_A):
                    pltpu.async_copy(ones_v, dacc.at[idxb.at[b, 2]], sem_s,
                                     add=True)
                for b in range(_NB_A):
                    pltpu.make_async_copy(ones_v, dacc.at[idxb.at[b, 2]],
                                          sem_s).wait()

            for b in range(_NB_A):
                pltpu.make_async_copy(rows.at[b], acc.at[idxb.at[b, 1]],
                                      sem_s).wait()

            @pl.when(k < _NSUP_A - 1)
            def _next_idx():
                pltpu.sync_copy(
                    sqd.at[pl.ds(row_base + (k + 1) * _NB_A, _NB_A)], idxb)
            return carry

        lax.fori_loop(0, _NSUP_A, _super, 0)

        for i in range(_LEFT_A):
            @pl.when(s == 8 * i)
            def _tail():
                pltpu.sync_copy(sqd.at[pl.ds(_RPT_MAIN_A * _NS + i, 1)],
                                idxb.at[pl.ds(0, 1)])
                _emb_gather(0, rows.at[0])
                pltpu.make_async_copy(embL.at[idxb.at[0, 0]], rows.at[0],
                                      sem_g).wait()
                pltpu.sync_copy(rows.at[0], acc.at[idxb.at[0, 1]], add=True)

                @pl.when(do_deg)
                def _degt():
                    pltpu.sync_copy(ones_v, dacc.at[idxb.at[0, 2]], add=True)

        plsc.subcore_barrier()

        pltpu.sync_copy(acc.at[pl.ds(s * _RPT_A, _RPT_A)],
                        xagg_out.at[c, pl.ds(s * _RPT_A, _RPT_A)])
        pltpu.sync_copy(dacc.at[pl.ds(s * _DPT_A, _DPT_A)],
                        deg_out.at[c, pl.ds(s * _DPT_A, _DPT_A)])

    return phase_a


_RPC = _NROW // _NC
_NB_C = 3
_RPT_MAIN_C = 39
_NSUP_C = _RPT_MAIN_C // _NB_C
_RPT_C = _UP // _NS


def _make_phase_c():
    mesh = plsc.VectorSubcoreMesh(
        core_axis_name="c", subcore_axis_name="s",
        num_cores=_NC, num_subcores=_NS)

    @functools.partial(
        pl.kernel,
        out_type=(jax.ShapeDtypeStruct((_NC, _UP, 128), jnp.float32),
                  jax.ShapeDtypeStruct((_NC, _UP, 8), jnp.float32)),
        mesh=mesh,
        compiler_params=pltpu.CompilerParams(use_tc_tiling_on_sc=False),
        scratch_types=[
            pltpu.VMEM_SHARED((_UP, 128), jnp.float32),
            pltpu.VMEM_SHARED((_UP, 8), jnp.float32),
            pltpu.VMEM((_NB_C, 2, _CH), jnp.int32),
            pltpu.VMEM((_NB_C, _CH, 128), jnp.float32),
            pltpu.VMEM((_CH, 8), jnp.float32),
            pltpu.VMEM((_CH, 8), jnp.float32),
            pltpu.SemaphoreType.DMA,
            pltpu.SemaphoreType.DMA,
        ],
    )
    def phase_c(xc, sc2, zeros128, zeros8, ones8,
                sum_out, cnt_out,
                acc, cacc, idxb, rows, zb8, ones_v, sem_g, sem_s):
        c = lax.axis_index("c")
        s = lax.axis_index("s")

        zb128 = rows.at[0]
        pltpu.sync_copy(zeros128, zb128)
        pltpu.sync_copy(zeros8, zb8)
        pltpu.sync_copy(ones8, ones_v)

        r0 = s * _RPT_C
        nz = _RPT_C // _CH
        def _za(i, carry):
            pltpu.sync_copy(zb128, acc.at[pl.ds(r0 + i * _CH, _CH)])
            pltpu.sync_copy(zb8, cacc.at[pl.ds(r0 + i * _CH, _CH)])
            return carry
        lax.fori_loop(0, nz, _za, 0)
        rem = _RPT_C - nz * _CH
        if rem:
            pltpu.sync_copy(zb128.at[pl.ds(0, rem)],
                            acc.at[pl.ds(r0 + nz * _CH, rem)])
            pltpu.sync_copy(zb8.at[pl.ds(0, rem)],
                            cacc.at[pl.ds(r0 + nz * _CH, rem)])

        plsc.subcore_barrier()

        row_base = c * _RPC + s * _RPT_MAIN_C
        pltpu.sync_copy(sc2.at[pl.ds(row_base, _NB_C)], idxb)

        def _super(k, carry):
            for b in range(_NB_C):
                pltpu.async_copy(xc.at[idxb.at[b, 0]], rows.at[b], sem_g)
            for b in range(_NB_C):
                pltpu.make_async_copy(xc.at[idxb.at[b, 0]], rows.at[b],
                                      sem_g).wait()
            for b in range(_NB_C):
                pltpu.async_copy(rows.at[b], acc.at[idxb.at[b, 1]], sem_s,
                                 add=True)
                pltpu.async_copy(ones_v, cacc.at[idxb.at[b, 1]], sem_s,
                                 add=True)
            for b in range(_NB_C):
                pltpu.make_async_copy(rows.at[b], acc.at[idxb.at[b, 1]],
                                      sem_s).wait()
                pltpu.make_async_copy(ones_v, cacc.at[idxb.at[b, 1]],
                                      sem_s).wait()

            @pl.when(k < _NSUP_C - 1)
            def _next_idx():
                pltpu.sync_copy(
                    sc2.at[pl.ds(row_base + (k + 1) * _NB_C, _NB_C)], idxb)
            return carry

        lax.fori_loop(0, _NSUP_C, _super, 0)

        @pl.when(s == 0)
        def _tail():
            pltpu.sync_copy(sc2.at[pl.ds(c * _RPC + _RPT_MAIN_C * _NS, 1)],
                            idxb.at[pl.ds(0, 1)])
            pltpu.async_copy(xc.at[idxb.at[0, 0]], rows.at[0], sem_g).wait()
            pltpu.sync_copy(rows.at[0], acc.at[idxb.at[0, 1]], add=True)
            pltpu.sync_copy(ones_v, cacc.at[idxb.at[0, 1]], add=True)

        plsc.subcore_barrier()

        pltpu.sync_copy(acc.at[pl.ds(s * _RPT_C, _RPT_C)],
                        sum_out.at[c, pl.ds(s * _RPT_C, _RPT_C)])
        pltpu.sync_copy(cacc.at[pl.ds(s * _RPT_C, _RPT_C)],
                        cnt_out.at[c, pl.ds(s * _RPT_C, _RPT_C)])

    return phase_c


_BB = 1000


def _pb_body(emb_ref, xaL_ref, xaR_ref, deg_ref, w1f_ref, b1f_ref, w2f_ref,
             w1n_ref, b1n_ref, w2nf_ref, hb_ref, g1_ref, bb1_ref,
             g2_ref, bb2_ref, out_ref):
    x = emb_ref[...]
    cf = _leaky(jax.lax.dot(x, w1f_ref[...],
                            preferred_element_type=jnp.float32)
                + b1f_ref[...])
    deg = jnp.maximum(deg_ref[0, :, 0:1] + deg_ref[1, :, 0:1], 1.0)

    agg = jnp.zeros((_BB, _HID), jnp.float32)
    for r in range(_RELS):
        xa = jnp.concatenate([xaL_ref[r], xaR_ref[r]], axis=-1)
        s2 = jax.lax.dot(xa, w2f_ref[r],
                         preferred_element_type=jnp.float32)
        for m in range(_MEM):
            agg += cf[:, r * _MEM + m:r * _MEM + m + 1] * \
                s2[:, m * _HID:(m + 1) * _HID]
    agg = agg / deg

    mu = jnp.mean(agg, axis=-1, keepdims=True)
    var = jnp.mean((agg - mu) ** 2, axis=-1, keepdims=True)
    nr = (agg - mu) * jax.lax.rsqrt(var + _EPS) * g1_ref[...] \
        + bb1_ref[...] + hb_ref[...]

    cn = _leaky(jax.lax.dot(x, w1n_ref[...],
                            preferred_element_type=jnp.float32)
                + b1n_ref[...])
    t = jax.lax.dot(x, w2nf_ref[...],
                    preferred_element_type=jnp.float32)
    for m in range(_MEM):
        nr += cn[:, m:m + 1] * t[:, m * _HID:(m + 1) * _HID]
    x1 = _leaky(nr)

    xcat = jnp.concatenate([x, x1], axis=-1)
    mu2 = jnp.mean(xcat, axis=-1, keepdims=True)
    var2 = jnp.mean((xcat - mu2) ** 2, axis=-1, keepdims=True)
    out_ref[...] = (xcat - mu2) * jax.lax.rsqrt(var2 + _EPS) * g2_ref[...] \
        + bb2_ref[...]


def _phase_b(emb, xaL, xaR, deg, w1f, b1f, w2f, w1n, b1n, w2nf,
             hb, g1, bb1, g2, bb2):
    grid = (_N_NODES // _BB,)
    return pl.pallas_call(
        _pb_body,
        grid=grid,
        in_specs=[
            pl.BlockSpec((_BB, _HID), lambda i: (i, 0)),
            pl.BlockSpec((_RELS, _BB, 32), lambda i: (0, i, 0)),
            pl.BlockSpec((_RELS, _BB, 32), lambda i: (0, i, 0)),
            pl.BlockSpec((_NC, _BB, 8), lambda i: (0, i, 0)),
            pl.BlockSpec((_HID, _RELS * _MEM), lambda i: (0, 0)),
            pl.BlockSpec((1, _RELS * _MEM), lambda i: (0, 0)),
            pl.BlockSpec((_RELS, _HID, _MEM * _HID), lambda i: (0, 0, 0)),
            pl.BlockSpec((_HID, _MEM), lambda i: (0, 0)),
            pl.BlockSpec((1, _MEM), lambda i: (0, 0)),
            pl.BlockSpec((_HID, _MEM * _HID), lambda i: (0, 0)),
            pl.BlockSpec((1, _HID), lambda i: (0, 0)),
            pl.BlockSpec((1, _HID), lambda i: (0, 0)),
            pl.BlockSpec((1, _HID), lambda i: (0, 0)),
            pl.BlockSpec((1, 2 * _HID), lambda i: (0, 0)),
            pl.BlockSpec((1, 2 * _HID), lambda i: (0, 0)),
        ],
        out_specs=pl.BlockSpec((_BB, 2 * _HID), lambda i: (i, 0)),
        out_shape=jax.ShapeDtypeStruct((_N_NODES, 2 * _HID), jnp.float32),
    )(emb, xaL, xaR, deg, w1f, b1f, w2f, w1n, b1n, w2nf, hb, g1, bb1, g2, bb2)


_BD = 1000


def _pd_body(xc_ref, s_ref, c_ref, out_ref):
    cnt = c_ref[0, :, 0:1] + c_ref[1, :, 0:1] + 1.0
    out_ref[...] = (s_ref[0] + s_ref[1] + xc_ref[...]) / cnt


def _phase_d(xc, sums, cnts):
    grid = (_N_USER // _BD,)
    return pl.pallas_call(
        _pd_body,
        grid=grid,
        in_specs=[
            pl.BlockSpec((_BD, 2 * _HID), lambda i: (i, 0)),
            pl.BlockSpec((_NC, _BD, 2 * _HID), lambda i: (0, i, 0)),
            pl.BlockSpec((_NC, _BD, 8), lambda i: (0, i, 0)),
        ],
        out_specs=pl.BlockSpec((_BD, 2 * _HID), lambda i: (i, 0)),
        out_shape=jax.ShapeDtypeStruct((_N_USER, 2 * _HID), jnp.float32),
    )(xc, sums, cnts)


_make_phase_a = functools.cache(_make_phase_a)
_make_phase_c = functools.cache(_make_phase_c)


def kernel(emb, W1, b1, W2, W1n, b1n, W2n, h_bias, ln1_g, ln1_b,
           ln2_g, ln2_b, edge_index, edge_type):
    src = edge_index[0].astype(jnp.int32)
    dst = edge_index[1].astype(jnp.int32)
    et = edge_type.astype(jnp.int32)

    q = et * _NP + dst
    mask = (et == 0) & (src < _N_USER) & (dst < _N_USER)
    cu = jnp.where(mask, dst, _N_USER)

    sqd = jnp.stack([src.reshape(_NROW, _CH), q.reshape(_NROW, _CH),
                     dst.reshape(_NROW, _CH)], axis=1)
    sc2 = jnp.stack([src.reshape(_NROW, _CH), cu.reshape(_NROW, _CH)],
                    axis=1)

    embL = emb[:, :32]
    embR = emb[:, 32:]
    zeros32 = jnp.zeros((_CH, 32), jnp.float32)
    zeros8 = jnp.zeros((_CH, 8), jnp.float32)
    zeros128 = jnp.zeros((_CH, 128), jnp.float32)
    ones8 = jnp.ones((_CH, 8), jnp.float32)

    xagg, deg = _make_phase_a()(embL, embR, sqd, zeros32, zeros8, ones8)
    xaL = xagg[0].reshape(_RELS, _NP, 32)
    xaR = xagg[1].reshape(_RELS, _NP, 32)

    w1f = W1.transpose(1, 0, 2).reshape(_HID, _RELS * _MEM)
    b1f = b1.reshape(1, _RELS * _MEM)
    w2f = W2.transpose(0, 3, 1, 2).reshape(_RELS, _HID, _MEM * _HID)
    w2nf = W2n.transpose(2, 0, 1).reshape(_HID, _MEM * _HID)
    b1nf = b1n.reshape(1, _MEM)
    hb = h_bias.reshape(1, _HID)
    g1 = ln1_g.reshape(1, _HID)
    bb1 = ln1_b.reshape(1, _HID)
    g2 = ln2_g.reshape(1, 2 * _HID)
    bb2 = ln2_b.reshape(1, 2 * _HID)

    xc = _phase_b(emb, xaL, xaR, deg, w1f, b1f, w2f, W1n, b1nf, w2nf,
                  hb, g1, bb1, g2, bb2)

    sums, cnts = _make_phase_c()(xc, sc2, zeros128, zeros8, ones8)
    user_pool = _phase_d(xc, sums, cnts)

    return (xc, user_pool)
